# baseline (speedup 1.0000x reference)
import functools

import jax
import jax.numpy as jnp
from jax import lax
from jax.experimental import pallas as pl
from jax.experimental.pallas import tpu as pltpu
from jax.experimental.pallas import tpu_sc as plsc

CHUNK = 256
LANES = 16
GRP = CHUNK // LANES


@functools.lru_cache(maxsize=None)
def _build_gather(V, D, J, S):
    info = plsc.get_sparse_core_info()
    NC, NS = info.num_cores, info.num_subcores
    NW = NC * NS
    B = J * S
    cpj = S // CHUNK
    n_chunks = B // CHUNK
    assert n_chunks % (2 * NW) == 0
    cpw = n_chunks // NW
    assert cpj & (cpj - 1) == 0
    cpj_shift = cpj.bit_length() - 1
    PITCH = CHUNK + 1
    mesh = plsc.VectorSubcoreMesh(core_axis_name="c", subcore_axis_name="s")

    @functools.partial(
        pl.kernel,
        mesh=mesh,
        out_type=jax.ShapeDtypeStruct((J, D, S), jnp.float32),
        scratch_types=[
            [pltpu.VMEM((CHUNK,), jnp.int32) for _ in range(2)],
            [pltpu.VMEM((CHUNK, D), jnp.float32) for _ in range(2)],
            [pltpu.VMEM((D, PITCH), jnp.float32) for _ in range(2)],
            [pltpu.SemaphoreType.DMA for _ in range(2)],
            [pltpu.SemaphoreType.DMA for _ in range(2)],
        ],
        compiler_params=pltpu.CompilerParams(
            use_tc_tiling_on_sc=False, needs_layout_passes=False),
    )
    def gather_k(table_hbm, idx_hbm, out_hbm, idx_v,
                 gbufs, tbufs, gsems, ssems):
        wid = lax.axis_index("s") * NC + lax.axis_index("c")
        c0 = wid * cpw
        i16 = lax.iota(jnp.int32, LANES)

        def load_indices(c, b):
            pltpu.sync_copy(idx_hbm.at[pl.ds(c * CHUNK, CHUNK)], idx_v[b])

        def fire_gather(b):
            pltpu.make_async_copy(
                table_hbm.at[idx_v[b]], gbufs[b], gsems[b]).start()

        def wait_gather(b):
            pltpu.make_async_copy(
                table_hbm.at[idx_v[b]], gbufs[b], gsems[b]).wait()

        def store_cp(c, b):
            j = c >> cpj_shift
            s0 = (c & (cpj - 1)) * CHUNK
            return pltpu.make_async_copy(
                tbufs[b].at[:, pl.ds(0, CHUNK)],
                out_hbm.at[j, :, pl.ds(s0, CHUNK)], ssems[b])

        def transpose(b):
            gbuf = gbufs[b]
            tbuf = tbufs[b]

            def sgroup(g, carry):
                for i in range(LANES):
                    s = g * LANES + i
                    srow = jnp.full((LANES,), 0, jnp.int32) + s
                    for k in range(D // LANES):
                        colv = i16 + k * LANES
                        vals = plsc.load_gather(gbuf, [srow, colv])
                        plsc.store_scatter(
                            tbuf, [i16 + k * LANES, srow], vals)
                return carry

            lax.fori_loop(0, GRP, sgroup, 0)

        load_indices(c0, 0)
        fire_gather(0)
        load_indices(c0 + 1, 1)
        fire_gather(1)

        def group(t, carry):
            for b in range(2):
                i = 2 * t + b
                c = c0 + i
                wait_gather(b)

                @pl.when(i >= 2)
                def _():
                    store_cp(c - 2, b).wait()

                transpose(b)
                store_cp(c, b).start()

                @pl.when(i + 2 < cpw)
                def _():
                    load_indices(c + 2, b)
                    fire_gather(b)

            return carry

        lax.fori_loop(0, cpw // 2, group, 0)
        store_cp(c0 + cpw - 2, 0).wait()
        store_cp(c0 + cpw - 1, 1).wait()

    return gather_k


def kernel(input, weight):
    B0, B1 = input.shape
    V, D = weight.shape
    idx = input.T.reshape(-1).astype(jnp.int32)
    out = _build_gather(V, D, B1, B0)(weight, idx)
    return jnp.transpose(out, (2, 0, 1))


# speedup vs baseline: 1.0026x; 1.0026x over previous
import functools

import jax
import jax.numpy as jnp
from jax import lax
from jax.experimental import pallas as pl
from jax.experimental.pallas import tpu as pltpu
from jax.experimental.pallas import tpu_sc as plsc

CHUNK = 256
LANES = 16
GRP = CHUNK // LANES


@functools.lru_cache(maxsize=None)
def _build_gather(V, D, J, S):
    info = plsc.get_sparse_core_info()
    NC, NS = info.num_cores, info.num_subcores
    NW = NC * NS
    B = J * S
    cpj = S // CHUNK
    n_chunks = B // CHUNK
    assert n_chunks % (2 * NW) == 0
    cpw = n_chunks // NW
    assert cpj & (cpj - 1) == 0
    cpj_shift = cpj.bit_length() - 1
    PITCH = CHUNK + 1
    mesh = plsc.VectorSubcoreMesh(core_axis_name="c", subcore_axis_name="s")

    @functools.partial(
        pl.kernel,
        mesh=mesh,
        out_type=jax.ShapeDtypeStruct((J * D, S), jnp.float32),
        scratch_types=[
            [pltpu.VMEM((CHUNK,), jnp.int32) for _ in range(2)],
            [pltpu.VMEM((CHUNK, D), jnp.float32) for _ in range(2)],
            [pltpu.VMEM((D, PITCH), jnp.float32) for _ in range(2)],
            [pltpu.SemaphoreType.DMA for _ in range(2)],
            [pltpu.SemaphoreType.DMA for _ in range(2)],
        ],
        compiler_params=pltpu.CompilerParams(
            use_tc_tiling_on_sc=False, needs_layout_passes=False),
    )
    def gather_k(table_hbm, idx_hbm, out_hbm, idx_v,
                 gbufs, tbufs, gsems, ssems):
        wid = lax.axis_index("s") * NC + lax.axis_index("c")
        c0 = wid * cpw
        i16 = lax.iota(jnp.int32, LANES)

        def load_indices(c, b):
            pltpu.sync_copy(idx_hbm.at[pl.ds(c * CHUNK, CHUNK)], idx_v[b])

        def fire_gather(b):
            pltpu.make_async_copy(
                table_hbm.at[idx_v[b]], gbufs[b], gsems[b]).start()

        def wait_gather(b):
            pltpu.make_async_copy(
                table_hbm.at[idx_v[b]], gbufs[b], gsems[b]).wait()

        def store_cp(c, b):
            j = c >> cpj_shift
            s0 = (c & (cpj - 1)) * CHUNK
            return pltpu.make_async_copy(
                tbufs[b].at[:, pl.ds(0, CHUNK)],
                out_hbm.at[pl.ds(j * D, D), pl.ds(s0, CHUNK)], ssems[b])

        def transpose(b):
            gbuf = gbufs[b]
            tbuf = tbufs[b]

            def sgroup(g, carry):
                for i in range(LANES):
                    s = g * LANES + i
                    scol = jnp.full((LANES,), 0, jnp.int32) + s
                    for k in range(D // LANES):
                        vals = plsc.load_gather(
                            gbuf.at[s], [i16 + k * LANES])
                        plsc.store_scatter(
                            tbuf, [i16 + k * LANES, scol], vals)
                return carry

            lax.fori_loop(0, GRP, sgroup, 0)

        load_indices(c0, 0)
        fire_gather(0)
        load_indices(c0 + 1, 1)
        fire_gather(1)

        def group(t, carry):
            for b in range(2):
                i = 2 * t + b
                c = c0 + i
                wait_gather(b)

                @pl.when(i >= 2)
                def _():
                    store_cp(c - 2, b).wait()

                transpose(b)
                store_cp(c, b).start()

                @pl.when(i + 2 < cpw)
                def _():
                    load_indices(c + 2, b)
                    fire_gather(b)

            return carry

        lax.fori_loop(0, cpw // 2, group, 0)
        store_cp(c0 + cpw - 2, 0).wait()
        store_cp(c0 + cpw - 1, 1).wait()

    return gather_k


def kernel(input, weight):
    B0, B1 = input.shape
    V, D = weight.shape
    idx = input.T.reshape(-1).astype(jnp.int32)
    out = _build_gather(V, D, B1, B0)(weight, idx)
    return jnp.transpose(out.reshape(B1, D, B0), (2, 0, 1))


# speedup vs baseline: 1.2412x; 1.2380x over previous
import functools

import jax
import jax.numpy as jnp
from jax import lax
from jax.experimental import pallas as pl
from jax.experimental.pallas import tpu as pltpu
from jax.experimental.pallas import tpu_sc as plsc

CHUNK = 256
LANES = 16
GRP = CHUNK // LANES


@functools.lru_cache(maxsize=None)
def _build_gather(V, D, J, S):
    info = plsc.get_sparse_core_info()
    NC, NS = info.num_cores, info.num_subcores
    NW = NC * NS
    B = J * S
    cpj = S // CHUNK
    n_chunks = B // CHUNK
    assert n_chunks % (2 * NW) == 0
    cpw = n_chunks // NW
    assert cpj & (cpj - 1) == 0
    cpj_shift = cpj.bit_length() - 1
    PITCH = CHUNK + 1
    mesh = plsc.VectorSubcoreMesh(core_axis_name="c", subcore_axis_name="s")

    @functools.partial(
        pl.kernel,
        mesh=mesh,
        out_type=jax.ShapeDtypeStruct((J * D, S), jnp.float32),
        scratch_types=[
            [pltpu.VMEM((CHUNK,), jnp.int32) for _ in range(2)],
            [pltpu.VMEM((CHUNK, D), jnp.float32) for _ in range(2)],
            [pltpu.VMEM((D, PITCH), jnp.float32) for _ in range(2)],
            [pltpu.SemaphoreType.DMA for _ in range(2)],
            [pltpu.SemaphoreType.DMA for _ in range(2)],
        ],
        compiler_params=pltpu.CompilerParams(
            use_tc_tiling_on_sc=False, needs_layout_passes=False),
    )
    def gather_k(table_hbm, idx_hbm, out_hbm, idx_v,
                 gbufs, tbufs, gsems, ssems):
        wid = lax.axis_index("s") * NC + lax.axis_index("c")
        c0 = wid * cpw
        i16 = lax.iota(jnp.int32, LANES)

        def load_indices(c, b):
            pltpu.sync_copy(idx_hbm.at[pl.ds(c * CHUNK, CHUNK)], idx_v[b])

        def fire_gather(b):
            pltpu.make_async_copy(
                table_hbm.at[idx_v[b]], gbufs[b], gsems[b]).start()

        def wait_gather(b):
            pltpu.make_async_copy(
                table_hbm.at[idx_v[b]], gbufs[b], gsems[b]).wait()

        def store_cp(c, b):
            j = c >> cpj_shift
            s0 = (c & (cpj - 1)) * CHUNK
            return pltpu.make_async_copy(
                tbufs[b].at[:, pl.ds(0, CHUNK)],
                out_hbm.at[pl.ds(j * D, D), pl.ds(s0, CHUNK)], ssems[b])

        def transpose(b):
            gbuf = gbufs[b]
            tbuf = tbufs[b]

            @plsc.parallel_loop(0, GRP)
            def sgroup(g):
                for i in range(LANES):
                    s = g * LANES + i
                    scol = jnp.full((LANES,), 0, jnp.int32) + s
                    for k in range(D // LANES):
                        vals = plsc.load_gather(
                            gbuf.at[s], [i16 + k * LANES])
                        plsc.store_scatter(
                            tbuf, [i16 + k * LANES, scol], vals)

        load_indices(c0, 0)
        fire_gather(0)
        load_indices(c0 + 1, 1)
        fire_gather(1)

        def group(t, carry):
            for b in range(2):
                i = 2 * t + b
                c = c0 + i
                wait_gather(b)

                @pl.when(i >= 2)
                def _():
                    store_cp(c - 2, b).wait()

                transpose(b)
                store_cp(c, b).start()

                @pl.when(i + 2 < cpw)
                def _():
                    load_indices(c + 2, b)
                    fire_gather(b)

            return carry

        lax.fori_loop(0, cpw // 2, group, 0)
        store_cp(c0 + cpw - 2, 0).wait()
        store_cp(c0 + cpw - 1, 1).wait()

    return gather_k


def kernel(input, weight):
    B0, B1 = input.shape
    V, D = weight.shape
    idx = input.T.reshape(-1).astype(jnp.int32)
    out = _build_gather(V, D, B1, B0)(weight, idx)
    return jnp.transpose(out.reshape(B1, D, B0), (2, 0, 1))
